# baseline (speedup 1.0000x reference)
import functools

import jax
import jax.numpy as jnp
from jax import lax
from jax.experimental import pallas as pl
from jax.experimental.pallas import tpu as pltpu
from jax.experimental.pallas import tpu_sc as plsc

NC = 2
NS = 16
NW = NC * NS
L = 16
D = 32
PACK = 4
CHUNK = 128


def _sc_kernel(b_per_w, n_chunks, uid_hbm, mid_hbm, uemb_hbm, memb_hbm,
               ubias_hbm, mbias_hbm, gbias_hbm, out_hbm,
               uid_v, mid_v, gu_v, gm_v, ubuf_v, mbuf_v, ub_v, mb_v, gb_v,
               out_v, sem, bsem):
    wid = lax.axis_index("s") * NC + lax.axis_index("c")
    base = wid * b_per_w

    pltpu.sync_copy(uid_hbm.at[pl.ds(base, b_per_w)], uid_v)
    pltpu.sync_copy(mid_hbm.at[pl.ds(base, b_per_w)], mid_v)
    pltpu.sync_copy(gbias_hbm, gb_v)

    bias_copies = []
    for j in range(n_chunks):
        rows = pl.ds(j * CHUNK, CHUNK)
        bias_copies.append(
            pltpu.async_copy(ubias_hbm.at[uid_v.at[rows]], ub_v.at[rows], bsem))
        bias_copies.append(
            pltpu.async_copy(mbias_hbm.at[mid_v.at[rows]], mb_v.at[rows], bsem))

    @pl.loop(0, b_per_w, step=L)
    def _(i):
        s = pl.ds(i, L)
        gu_v[s] = lax.shift_right_logical(uid_v[s], 2)
        gm_v[s] = lax.shift_right_logical(mid_v[s], 2)

    def fire(j, buf):
        rows = pl.ds(j * CHUNK, CHUNK)
        return (pltpu.async_copy(uemb_hbm.at[gu_v.at[rows]], ubuf_v.at[buf], sem),
                pltpu.async_copy(memb_hbm.at[gm_v.at[rows]], mbuf_v.at[buf], sem))

    gb = gb_v[pl.ds(0, L)]
    pending = fire(0, 0)
    for j in range(n_chunks):
        nxt = fire(j + 1, (j + 1) % 2) if j + 1 < n_chunks else None
        pending[0].wait()
        pending[1].wait()
        if j == 0:
            for c in bias_copies:
                c.wait()
        ubuf = ubuf_v.at[j % 2]
        mbuf = mbuf_v.at[j % 2]

        @pl.loop(0, CHUNK, step=L)
        def _(i, j=j, ubuf=ubuf, mbuf=mbuf):
            gi = j * CHUNK + i
            sg = pl.ds(gi, L)
            rows16 = i + lax.iota(jnp.int32, L)
            cu = (uid_v[sg] & 3) * D
            cm = (mid_v[sg] & 3) * D
            acc0 = ub_v[sg] + mb_v[sg] + gb
            acc1 = jnp.zeros((L,), jnp.float32)
            acc2 = jnp.zeros((L,), jnp.float32)
            acc3 = jnp.zeros((L,), jnp.float32)
            for d in range(0, D, 4):
                acc0 = acc0 + (plsc.load_gather(ubuf, [rows16, cu + d])
                               * plsc.load_gather(mbuf, [rows16, cm + d]))
                acc1 = acc1 + (plsc.load_gather(ubuf, [rows16, cu + (d + 1)])
                               * plsc.load_gather(mbuf, [rows16, cm + (d + 1)]))
                acc2 = acc2 + (plsc.load_gather(ubuf, [rows16, cu + (d + 2)])
                               * plsc.load_gather(mbuf, [rows16, cm + (d + 2)]))
                acc3 = acc3 + (plsc.load_gather(ubuf, [rows16, cu + (d + 3)])
                               * plsc.load_gather(mbuf, [rows16, cm + (d + 3)]))
            out_v[sg] = (acc0 + acc1) + (acc2 + acc3)

        pending = nxt

    pltpu.sync_copy(out_v, out_hbm.at[pl.ds(base, b_per_w)])


def _pack_rows(table, one):
    return table.reshape(table.shape[0] // PACK, PACK * table.shape[1]) * one


def kernel(user_ids, movie_ids, user_emb_table, movie_emb_table,
           user_bias_table, movie_bias_table, global_bias):
    B = user_ids.shape[0]
    V, D_ = user_emb_table.shape
    assert D_ == D and V % PACK == 0 and B % (NW * CHUNK) == 0
    b_per_w = B // NW
    n_chunks = b_per_w // CHUNK

    uid = user_ids.astype(jnp.int32)
    mid = movie_ids.astype(jnp.int32)
    one = jnp.exp(0.0 * global_bias[0].astype(jnp.float32))
    uemb = _pack_rows(user_emb_table, one)
    memb = _pack_rows(movie_emb_table, one)
    ubias = user_bias_table.reshape(-1)
    mbias = movie_bias_table.reshape(-1)
    gb128 = jnp.broadcast_to(global_bias.astype(jnp.float32), (128,))

    mesh = plsc.VectorSubcoreMesh(core_axis_name="c", subcore_axis_name="s")
    body = functools.partial(_sc_kernel, b_per_w, n_chunks)
    run = pl.kernel(
        body,
        out_type=jax.ShapeDtypeStruct((B,), jnp.float32),
        mesh=mesh,
        compiler_params=pltpu.CompilerParams(needs_layout_passes=False),
        scratch_types=[
            pltpu.VMEM((b_per_w,), jnp.int32),
            pltpu.VMEM((b_per_w,), jnp.int32),
            pltpu.VMEM((b_per_w,), jnp.int32),
            pltpu.VMEM((b_per_w,), jnp.int32),
            pltpu.VMEM((2, CHUNK, PACK * D), jnp.float32),
            pltpu.VMEM((2, CHUNK, PACK * D), jnp.float32),
            pltpu.VMEM((b_per_w,), jnp.float32),
            pltpu.VMEM((b_per_w,), jnp.float32),
            pltpu.VMEM((128,), jnp.float32),
            pltpu.VMEM((b_per_w,), jnp.float32),
            pltpu.SemaphoreType.DMA,
            pltpu.SemaphoreType.DMA,
        ],
    )
    return run(uid, mid, uemb, memb, ubias, mbias, gb128)


# speedup vs baseline: 1.6663x; 1.6663x over previous
import functools

import jax
import jax.numpy as jnp
from jax import lax
from jax.experimental import pallas as pl
from jax.experimental.pallas import tpu as pltpu
from jax.experimental.pallas import tpu_sc as plsc

NC = 2
NS = 16
NW = NC * NS
L = 16
D = 32
PACK = 4
CHUNK = 128


def _sc_kernel(b_per_w, n_chunks, uid_hbm, mid_hbm, uemb_hbm, memb_hbm,
               ubias_hbm, mbias_hbm, gbias_hbm, out_hbm,
               uid_v, mid_v, gu_v, gm_v, ubuf_v, mbuf_v, ub_v, mb_v, gb_v,
               out_v, sem, bsem):
    wid = lax.axis_index("s") * NC + lax.axis_index("c")
    base = wid * b_per_w

    pltpu.sync_copy(uid_hbm.at[pl.ds(base, b_per_w)], uid_v)
    pltpu.sync_copy(mid_hbm.at[pl.ds(base, b_per_w)], mid_v)
    pltpu.sync_copy(gbias_hbm, gb_v)

    bias_copies = []
    for j in range(n_chunks):
        rows = pl.ds(j * CHUNK, CHUNK)
        bias_copies.append(
            pltpu.async_copy(ubias_hbm.at[uid_v.at[rows]], ub_v.at[rows], bsem))
        bias_copies.append(
            pltpu.async_copy(mbias_hbm.at[mid_v.at[rows]], mb_v.at[rows], bsem))

    @pl.loop(0, b_per_w, step=L)
    def _(i):
        s = pl.ds(i, L)
        gu_v[s] = lax.shift_right_logical(uid_v[s], 2)
        gm_v[s] = lax.shift_right_logical(mid_v[s], 2)

    def fire(j, buf):
        rows = pl.ds(j * CHUNK, CHUNK)
        return (pltpu.async_copy(uemb_hbm.at[gu_v.at[rows]], ubuf_v.at[buf], sem),
                pltpu.async_copy(memb_hbm.at[gm_v.at[rows]], mbuf_v.at[buf], sem))

    gb = gb_v[pl.ds(0, L)]
    pending = fire(0, 0)
    for j in range(n_chunks):
        nxt = fire(j + 1, (j + 1) % 2) if j + 1 < n_chunks else None
        pending[0].wait()
        pending[1].wait()
        if j == 0:
            for c in bias_copies:
                c.wait()
        ubuf = ubuf_v.at[j % 2]
        mbuf = mbuf_v.at[j % 2]

        @pl.loop(0, CHUNK, step=L)
        def _(i, j=j, ubuf=ubuf, mbuf=mbuf):
            gi = j * CHUNK + i
            sg = pl.ds(gi, L)
            rows16 = i + lax.iota(jnp.int32, L)
            cu = (uid_v[sg] & 3) * D
            cm = (mid_v[sg] & 3) * D
            acc0 = ub_v[sg] + mb_v[sg] + gb
            acc1 = jnp.zeros((L,), jnp.float32)
            acc2 = jnp.zeros((L,), jnp.float32)
            acc3 = jnp.zeros((L,), jnp.float32)
            for d in range(0, D, 4):
                acc0 = acc0 + (plsc.load_gather(ubuf, [rows16, cu + d])
                               * plsc.load_gather(mbuf, [rows16, cm + d]))
                acc1 = acc1 + (plsc.load_gather(ubuf, [rows16, cu + (d + 1)])
                               * plsc.load_gather(mbuf, [rows16, cm + (d + 1)]))
                acc2 = acc2 + (plsc.load_gather(ubuf, [rows16, cu + (d + 2)])
                               * plsc.load_gather(mbuf, [rows16, cm + (d + 2)]))
                acc3 = acc3 + (plsc.load_gather(ubuf, [rows16, cu + (d + 3)])
                               * plsc.load_gather(mbuf, [rows16, cm + (d + 3)]))
            out_v[sg] = (acc0 + acc1) + (acc2 + acc3)

        pending = nxt

    pltpu.sync_copy(out_v, out_hbm.at[pl.ds(base, b_per_w)])


def _pack_rows(table):
    return table.reshape(table.shape[0] // PACK, PACK * table.shape[1])


def kernel(user_ids, movie_ids, user_emb_table, movie_emb_table,
           user_bias_table, movie_bias_table, global_bias):
    B = user_ids.shape[0]
    V, D_ = user_emb_table.shape
    assert D_ == D and V % PACK == 0 and B % (NW * CHUNK) == 0
    b_per_w = B // NW
    n_chunks = b_per_w // CHUNK

    uid = user_ids.astype(jnp.int32)
    mid = movie_ids.astype(jnp.int32)
    uemb = _pack_rows(user_emb_table)
    memb = _pack_rows(movie_emb_table)
    ubias = user_bias_table.reshape(-1)
    mbias = movie_bias_table.reshape(-1)
    gb128 = jnp.broadcast_to(global_bias.astype(jnp.float32), (128,))

    mesh = plsc.VectorSubcoreMesh(core_axis_name="c", subcore_axis_name="s")
    body = functools.partial(_sc_kernel, b_per_w, n_chunks)
    run = pl.kernel(
        body,
        out_type=jax.ShapeDtypeStruct((B,), jnp.float32),
        mesh=mesh,
        compiler_params=pltpu.CompilerParams(needs_layout_passes=False),
        scratch_types=[
            pltpu.VMEM((b_per_w,), jnp.int32),
            pltpu.VMEM((b_per_w,), jnp.int32),
            pltpu.VMEM((b_per_w,), jnp.int32),
            pltpu.VMEM((b_per_w,), jnp.int32),
            pltpu.VMEM((2, CHUNK, PACK * D), jnp.float32),
            pltpu.VMEM((2, CHUNK, PACK * D), jnp.float32),
            pltpu.VMEM((b_per_w,), jnp.float32),
            pltpu.VMEM((b_per_w,), jnp.float32),
            pltpu.VMEM((128,), jnp.float32),
            pltpu.VMEM((b_per_w,), jnp.float32),
            pltpu.SemaphoreType.DMA,
            pltpu.SemaphoreType.DMA,
        ],
    )
    return run(uid, mid, uemb, memb, ubias, mbias, gb128)
